# baseline (speedup 1.0000x reference)
import functools

import jax
import jax.numpy as jnp
from jax import lax
from jax.experimental import pallas as pl
from jax.experimental.pallas import tpu as pltpu
from jax.experimental.pallas import tpu_sc as plsc

NC = 2
NS = 16
NW = NC * NS
L = 16


@functools.lru_cache(maxsize=None)
def _build(B, V, F, D):
    BPW = B // NW
    n_groups = BPW // L

    mesh = plsc.VectorSubcoreMesh(core_axis_name="c", subcore_axis_name="s")

    @functools.partial(
        pl.kernel,
        mesh=mesh,
        out_type=jax.ShapeDtypeStruct((B, D), jnp.float32),
        compiler_params=pltpu.CompilerParams(needs_layout_passes=False),
        scratch_types=[
            pltpu.VMEM((BPW,), jnp.int32),
            pltpu.VMEM((BPW,), jnp.int32),
            pltpu.VMEM((BPW,), jnp.int32),
            pltpu.VMEM((F, D), jnp.float32),
            pltpu.VMEM((2, L, D), jnp.float32),
            pltpu.SemaphoreType.DMA,
            pltpu.SemaphoreType.DMA,
            pltpu.SemaphoreType.DMA,
        ],
    )
    def sc_kernel(fpi_hbm, fbi_hbm, val_hbm, fpm_hbm, fbt_hbm, out_hbm,
                  fpi_v, fbi_v, val_v, fb_stage, rows,
                  sem_g, sem_w, sem_m):
        wid = lax.axis_index("s") * NC + lax.axis_index("c")
        base = wid * BPW
        pltpu.sync_copy(fpi_hbm.at[pl.ds(base, BPW)], fpi_v)
        pltpu.sync_copy(fbi_hbm.at[pl.ds(base, BPW)], fbi_v)
        pltpu.sync_copy(val_hbm.at[pl.ds(base, BPW)], val_v)
        cp_stage = pltpu.async_copy(fbt_hbm, fb_stage, sem_w)

        lanes = lax.iota(jnp.int32, L)
        row_bytes = L * D * 4

        def gather(g, b):
            off = g * L
            pltpu.async_copy(fpm_hbm.at[fpi_v.at[pl.ds(off, L)]],
                             rows.at[b], sem_g)

        gather(0, 0)
        cp_stage.wait()

        def step(g, carry):
            off = g * L
            b = lax.rem(g, 2)
            bn = 1 - b

            @pl.when(g > 0)
            def _():
                pltpu.make_async_copy(rows.at[bn],
                                      out_hbm.at[pl.ds(base, L)],
                                      sem_w).wait()

            @pl.when(g + 1 < n_groups)
            def _():
                gather(jnp.minimum(g + 1, n_groups - 1), bn)

            pltpu.make_async_copy(fpm_hbm.at[fpi_v.at[pl.ds(off, L)]],
                                  rows.at[b], sem_g).wait()

            val16 = val_v[pl.ds(off, L)]
            fbi16 = fbi_v[pl.ds(off, L)]
            buf = rows.at[b]

            for e in range(L):
                @pl.when(val16[e] == 0)
                def _(e=e):
                    fbe = fbi16[e]

                    def cp(j, c):
                        buf[e, pl.ds(j * L, L)] = fb_stage[fbe,
                                                           pl.ds(j * L, L)]
                        return c

                    lax.fori_loop(0, D // L, cp, 0, unroll=8)

            pltpu.async_copy(buf, out_hbm.at[pl.ds(base + off, L)], sem_w)
            return carry

        lax.fori_loop(0, n_groups, step, 0)

        pltpu.make_async_copy(rows.at[lax.rem(jnp.int32(n_groups - 1), 2)],
                              out_hbm.at[pl.ds(base, L)], sem_w).wait()

    return sc_kernel


def kernel(fp_idx, fb_idx, is_valid, fingerprint_matrix, fallback_table):
    B = fp_idx.shape[0]
    D = fingerprint_matrix.shape[1]
    sc = _build(B, fingerprint_matrix.shape[0], fallback_table.shape[0], D)
    return sc(fp_idx.astype(jnp.int32),
              fb_idx.astype(jnp.int32),
              is_valid.astype(jnp.int32),
              fingerprint_matrix,
              fallback_table)
